# baseline (speedup 1.0000x reference)
import jax
from jax.experimental import pallas as pl
from jax.experimental.pallas import tpu as pltpu


def _copy_body(x_ref, o_ref, sem):
    copy = pltpu.make_async_copy(x_ref, o_ref, sem)
    copy.start()
    copy.wait()


def kernel(x):
    return pl.pallas_call(
        _copy_body,
        out_shape=jax.ShapeDtypeStruct(x.shape, x.dtype),
        in_specs=[pl.BlockSpec(memory_space=pl.ANY)],
        out_specs=pl.BlockSpec(memory_space=pl.ANY),
        scratch_shapes=[pltpu.SemaphoreType.DMA],
    )(x)


# speedup vs baseline: 12.5616x; 12.5616x over previous
import jax
from jax.experimental import pallas as pl
from jax.experimental.pallas import tpu as pltpu


def _copy_block(x_ref, o_ref):
    o_ref[...] = x_ref[...]


def kernel(x):
    b, c, h, w = x.shape
    rows, cols = b * c, h * w
    x2 = x.reshape(rows, cols)
    block_rows = rows
    for cand in range(rows, 0, -1):
        if rows % cand == 0 and cand * cols * x.dtype.itemsize <= 8 * 1024 * 1024:
            block_rows = cand
            break
    grid = rows // block_rows
    out = pl.pallas_call(
        _copy_block,
        out_shape=jax.ShapeDtypeStruct(x2.shape, x2.dtype),
        grid=(grid,),
        in_specs=[pl.BlockSpec((block_rows, cols), lambda i: (i, 0))],
        out_specs=pl.BlockSpec((block_rows, cols), lambda i: (i, 0)),
        compiler_params=pltpu.CompilerParams(dimension_semantics=("parallel",)),
    )(x2)
    return out.reshape(x.shape)


# speedup vs baseline: 49.0050x; 3.9012x over previous
import jax
from jax.experimental import pallas as pl
from jax.experimental.pallas import tpu as pltpu


def _copy_block(x_ref, o_ref):
    o_ref[...] = x_ref[...]


def kernel(x):
    b, c, h, w = x.shape
    blk_c = c
    for cand in range(c, 0, -1):
        if c % cand == 0 and cand * h * w * x.dtype.itemsize <= 8 * 1024 * 1024:
            blk_c = cand
            break
    grid = (b, c // blk_c)
    return pl.pallas_call(
        _copy_block,
        out_shape=jax.ShapeDtypeStruct(x.shape, x.dtype),
        grid=grid,
        in_specs=[pl.BlockSpec((1, blk_c, h, w), lambda i, j: (i, j, 0, 0))],
        out_specs=pl.BlockSpec((1, blk_c, h, w), lambda i, j: (i, j, 0, 0)),
        compiler_params=pltpu.CompilerParams(
            dimension_semantics=("parallel", "parallel")),
    )(x)
